# Initial kernel scaffold; baseline (speedup 1.0000x reference)
import jax
import jax.numpy as jnp
from jax.experimental import pallas as pl


def kernel(feat_x, feat_y):
    raise NotImplementedError("write your pallas kernel here")


# speedup vs baseline: 13.1588x; 13.1588x over previous
import functools

import jax
import jax.numpy as jnp
from jax.experimental import pallas as pl
from jax.experimental.pallas import tpu as pltpu

_TAU = 0.2
_K = 15


def _fused_topk_kernel(fx_ref, fy_ref, idx_ref, val_ref, *, ny, k, inv_tau):
    fx = fx_ref[0]
    fy = fy_ref[0]

    nx = jnp.sqrt(jnp.sum(fx * fx, axis=-1, keepdims=True))
    fxn = fx / jnp.maximum(nx, 1e-12)
    nyn = jnp.sqrt(jnp.sum(fy * fy, axis=-1, keepdims=True))
    fyn = fy / jnp.maximum(nyn, 1e-12)

    sim = jax.lax.dot_general(
        fxn, fyn, (((1,), (1,)), ((), ())),
        preferred_element_type=jnp.float32,
    ) * inv_tau

    r = sim.shape[0]
    iota = jax.lax.broadcasted_iota(jnp.int32, (r, ny), 1)
    neg = jnp.finfo(jnp.float32).min

    x = sim
    vals = []
    idxs = []
    for _ in range(k):
        m = jnp.max(x, axis=1, keepdims=True)
        hit = x == m
        ji = jnp.min(jnp.where(hit, iota, ny), axis=1, keepdims=True)
        vals.append(m)
        idxs.append(ji)
        x = jnp.where(iota == ji, neg, x)

    v = jnp.concatenate(vals, axis=1)
    i = jnp.concatenate(idxs, axis=1)

    e = jnp.exp(v - v[:, :1])
    sm = e / jnp.sum(e, axis=1, keepdims=True)

    idx_ref[0] = i
    val_ref[0] = sm


def kernel(feat_x, feat_y):
    b, nx, c = feat_x.shape
    ny = feat_y.shape[1]
    r = 256
    grid = (b, nx // r)

    body = functools.partial(
        _fused_topk_kernel, ny=ny, k=_K, inv_tau=1.0 / _TAU)

    idx, val = pl.pallas_call(
        body,
        grid=grid,
        in_specs=[
            pl.BlockSpec((1, r, c), lambda bi, i: (bi, i, 0)),
            pl.BlockSpec((1, ny, c), lambda bi, i: (bi, 0, 0)),
        ],
        out_specs=[
            pl.BlockSpec((1, r, _K), lambda bi, i: (bi, i, 0)),
            pl.BlockSpec((1, r, _K), lambda bi, i: (bi, i, 0)),
        ],
        out_shape=[
            jax.ShapeDtypeStruct((b, nx, _K), jnp.int32),
            jax.ShapeDtypeStruct((b, nx, _K), jnp.float32),
        ],
        compiler_params=pltpu.CompilerParams(
            dimension_semantics=("arbitrary", "arbitrary"),
        ),
    )(feat_x, feat_y)
    return idx, val
